# Initial kernel scaffold; baseline (speedup 1.0000x reference)
import functools

import jax
import jax.numpy as jnp
from jax import lax
from jax.experimental import pallas as pl
from jax.experimental.pallas import tpu as pltpu
from jax.experimental.pallas import tpu_sc as plsc

N = 10000
E = 160000
D = 256
H2 = 64
LANES = 16
NCORES = 2
NTILES = 16
ROWS_PER_TILE = N // NTILES
E_PER_TILE = E // NTILES
EB = 80
NB = E_PER_TILE // EB


def _mm_body(a_ref, w_ref, o_ref, *, relu):
    a = a_ref[...]
    if relu:
        a = jnp.maximum(a, 0.0)
    o_ref[...] = jnp.dot(a, w_ref[...], preferred_element_type=jnp.float32)


def _mm(a, w, relu=False, rows=1000):
    n, k = a.shape
    co = w.shape[1] // 64
    nr = n // rows
    return pl.pallas_call(
        functools.partial(_mm_body, relu=relu),
        grid=(co, nr),
        in_specs=[
            pl.BlockSpec((rows, k), lambda c, r: (r, 0)),
            pl.BlockSpec((k, 64), lambda c, r: (0, c)),
        ],
        out_specs=pl.BlockSpec((rows, 64), lambda c, r: (c * nr + r, 0)),
        out_shape=jax.ShapeDtypeStruct((co * n, 64), jnp.float32),
    )(a, w)


def _z_body(mu_ref, lv_ref, eps_ref, o_ref):
    o_ref[...] = eps_ref[...] * jnp.exp(lv_ref[...]) + mu_ref[...]


def _reparam(mu, logvar, eps, rows=1000):
    nr = N // rows
    spec = pl.BlockSpec((rows, H2), lambda r: (r, 0))
    return pl.pallas_call(
        _z_body,
        grid=(nr,),
        in_specs=[spec, spec, spec],
        out_specs=spec,
        out_shape=jax.ShapeDtypeStruct((N, H2), jnp.float32),
    )(mu, logvar, eps)


def _zzt_body(a_ref, b_ref, o_ref):
    o_ref[...] = lax.dot_general(
        a_ref[...], b_ref[...], (((1,), (1,)), ((), ())),
        preferred_element_type=jnp.float32)


def _zzt(z, rows=1000):
    nr = N // rows
    return pl.pallas_call(
        _zzt_body,
        grid=(nr, nr),
        in_specs=[
            pl.BlockSpec((rows, H2), lambda i, j: (i, 0)),
            pl.BlockSpec((rows, H2), lambda i, j: (j, 0)),
        ],
        out_specs=pl.BlockSpec((rows, rows), lambda i, j: (i, j)),
        out_shape=jax.ShapeDtypeStruct((N, N), jnp.float32),
    )(z, z)


def _spmm(sup_flat, src, dst, nchunks):
    fout = nchunks * 64
    mesh = plsc.VectorSubcoreMesh(core_axis_name="c", subcore_axis_name="s")
    zeros = jnp.zeros((N, 64), jnp.float32)

    @functools.partial(
        pl.kernel,
        mesh=mesh,
        out_type=jax.ShapeDtypeStruct((N, fout), jnp.float32),
        scratch_types=[
            pltpu.VMEM((EB,), jnp.int32),
            pltpu.VMEM((EB,), jnp.int32),
            pltpu.VMEM((EB, 64), jnp.float32),
            pltpu.VMEM_SHARED((N, 64), jnp.float32),
            pltpu.SemaphoreType.DMA,
        ],
    )
    def k(sup_hbm, src_hbm, dst_hbm, zer_hbm, out_hbm,
          src_v, dst_v, rows_v, acc, sem):
        cid = lax.axis_index("c")
        sid = lax.axis_index("s")
        row0 = sid * ROWS_PER_TILE
        ebase = sid * E_PER_TILE
        for half in range(nchunks // NCORES):
            c = half * NCORES + cid
            coff = c * N
            pltpu.sync_copy(zer_hbm.at[pl.ds(row0, ROWS_PER_TILE)],
                            acc.at[pl.ds(row0, ROWS_PER_TILE)])
            plsc.subcore_barrier()

            def block(b, carry):
                base = ebase + b * EB
                pltpu.sync_copy(src_hbm.at[pl.ds(base, EB)], src_v)
                pltpu.sync_copy(dst_hbm.at[pl.ds(base, EB)], dst_v)
                for i in range(EB // LANES):
                    sl = pl.ds(i * LANES, LANES)
                    src_v[sl] = src_v[sl] + coff
                pltpu.async_copy(sup_hbm.at[src_v], rows_v, sem).wait()
                pltpu.sync_copy(rows_v, acc.at[dst_v], add=True)
                return carry

            lax.fori_loop(0, NB, block, 0)
            plsc.subcore_barrier()
            pltpu.sync_copy(
                acc.at[pl.ds(row0, ROWS_PER_TILE)],
                out_hbm.at[pl.ds(row0, ROWS_PER_TILE), pl.ds(c * 64, 64)])

    return k(sup_flat, src, dst, zeros)


def kernel(x, edge_index, W1, W2, W3, Wd1, Wd2):
    src = edge_index[0]
    dst = edge_index[1]
    W23 = jnp.concatenate([W2, W3], axis=1)
    eps = jax.random.normal(jax.random.key(1), (N, H2), dtype=jnp.float32)

    sup1 = _mm(x, W1)
    agg1 = _spmm(sup1, src, dst, 4)
    sup23 = _mm(agg1, W23, relu=True)
    agg23 = _spmm(sup23, src, dst, 2)
    mu = agg23[:, :H2]
    logvar = agg23[:, H2:]
    z = _reparam(mu, logvar, eps)

    recon_adj = _zzt(z)

    supd1 = _mm(z, Wd1)
    aggd1 = _spmm(supd1, src, dst, 4)
    supd2 = _mm(aggd1, Wd2, relu=True)
    x_rec = _spmm(supd2, src, dst, 4)

    return (recon_adj, mu, logvar, z, x_rec)


# speedup vs baseline: 2.9410x; 2.9410x over previous
import functools

import jax
import jax.numpy as jnp
from jax import lax
from jax.experimental import pallas as pl
from jax.experimental.pallas import tpu as pltpu
from jax.experimental.pallas import tpu_sc as plsc

N = 10000
E = 160000
H2 = 64
LANES = 16
NCORES = 2
NTILES = 16
RPT = 624
RTAIL = N - RPT * NTILES
E_PER_TILE = E // NTILES
EB = 80
NB = E_PER_TILE // EB


def _mm_body_2d(a_ref, w_ref, o_ref, *, relu):
    a = a_ref[...]
    if relu:
        a = jnp.maximum(a, 0.0)
    o_ref[...] = jnp.dot(a, w_ref[0], preferred_element_type=jnp.float32)


def _mm_body_3d(a_ref, w_ref, o_ref, *, relu, ci, cw_in):
    acc = None
    for i in range(ci):
        a = a_ref[i]
        if relu:
            a = jnp.maximum(a, 0.0)
        p = jnp.dot(a, w_ref[0, i * cw_in:(i + 1) * cw_in, :],
                    preferred_element_type=jnp.float32)
        acc = p if acc is None else acc + p
    o_ref[...] = acc


def _mm(a, w, relu=False, cw_out=128, rows=1000):
    k, fo = w.shape
    co = fo // cw_out
    if a.ndim == 2:
        n = a.shape[0]
        body = functools.partial(_mm_body_2d, relu=relu)
        a_spec = pl.BlockSpec((rows, k), lambda c, r: (r, 0))
    else:
        ci, n, cw_in = a.shape
        body = functools.partial(_mm_body_3d, relu=relu, ci=ci, cw_in=cw_in)
        a_spec = pl.BlockSpec((ci, rows, cw_in), lambda c, r: (0, r, 0))
    nr = n // rows
    w3 = w.reshape(k, co, cw_out).transpose(1, 0, 2)
    return pl.pallas_call(
        body,
        grid=(co, nr),
        in_specs=[
            a_spec,
            pl.BlockSpec((1, k, cw_out), lambda c, r: (c, 0, 0)),
        ],
        out_specs=pl.BlockSpec((rows, cw_out), lambda c, r: (c * nr + r, 0)),
        out_shape=jax.ShapeDtypeStruct((co * n, cw_out), jnp.float32),
    )(a, w3)


def _mlz_body(p0_ref, p1_ref, eps_ref, mu_ref, lv_ref, z_ref):
    s = p0_ref[...] + p1_ref[...]
    mu = s[:, :H2]
    lv = s[:, H2:]
    mu_ref[...] = mu
    lv_ref[...] = lv
    z_ref[...] = eps_ref[...] * jnp.exp(lv) + mu


def _mlz(agg23p, eps, rows=1000):
    nr = N // rows
    ospec = pl.BlockSpec((rows, H2), lambda r: (r, 0))
    oshape = jax.ShapeDtypeStruct((N, H2), jnp.float32)
    return pl.pallas_call(
        _mlz_body,
        grid=(nr,),
        in_specs=[
            pl.BlockSpec((rows, 2 * H2), lambda r: (r, 0)),
            pl.BlockSpec((rows, 2 * H2), lambda r: (r + nr, 0)),
            ospec,
        ],
        out_specs=[ospec, ospec, ospec],
        out_shape=[oshape, oshape, oshape],
    )(agg23p, agg23p, eps)


def _zzt_body(a_ref, b_ref, o_ref):
    o_ref[...] = lax.dot_general(
        a_ref[...], b_ref[...], (((1,), (1,)), ((), ())),
        preferred_element_type=jnp.float32)


def _zzt(z, rows=400):
    nr = N // rows
    return pl.pallas_call(
        _zzt_body,
        grid=(nr,),
        in_specs=[
            pl.BlockSpec((rows, H2), lambda i: (i, 0)),
            pl.BlockSpec((N, H2), lambda i: (0, 0)),
        ],
        out_specs=pl.BlockSpec((rows, N), lambda i: (i, 0)),
        out_shape=jax.ShapeDtypeStruct((N, N), jnp.float32),
    )(z, z)


def _spmm(sup_flat, src, dst, split=False):
    mesh = plsc.VectorSubcoreMesh(core_axis_name="c", subcore_axis_name="s")
    zeros = jnp.zeros((N, 128), jnp.float32)
    eb = 40 if split else EB
    nb = (E // NCORES // NTILES // eb) if split else NB

    @functools.partial(
        pl.kernel,
        mesh=mesh,
        out_type=jax.ShapeDtypeStruct((NCORES * N, 128), jnp.float32),
        scratch_types=[
            pltpu.VMEM((eb,), jnp.int32),
            pltpu.VMEM((eb,), jnp.int32),
            pltpu.VMEM((eb, 128), jnp.float32),
            pltpu.VMEM_SHARED((N, 128), jnp.float32),
            pltpu.SemaphoreType.DMA,
        ],
    )
    def k(sup_hbm, src_hbm, dst_hbm, zer_hbm, out_hbm,
          src_v, dst_v, rows_v, acc, sem):
        cid = lax.axis_index("c")
        sid = lax.axis_index("s")
        row0 = pl.multiple_of(sid * RPT, 8)
        if split:
            ebase = cid * (E // NCORES) + sid * (E // NCORES // NTILES)
            coff = 0
        else:
            ebase = sid * E_PER_TILE
            coff = cid * N
        last = sid == NTILES - 1

        pltpu.sync_copy(zer_hbm.at[pl.ds(row0, RPT)], acc.at[pl.ds(row0, RPT)])

        @pl.when(last)
        def _():
            pltpu.sync_copy(zer_hbm.at[pl.ds(RPT * NTILES, RTAIL)],
                            acc.at[pl.ds(RPT * NTILES, RTAIL)])

        plsc.subcore_barrier()

        def block(b, carry):
            base = pl.multiple_of(ebase + b * eb, 8)
            pltpu.sync_copy(src_hbm.at[pl.ds(base, eb)], src_v)
            pltpu.sync_copy(dst_hbm.at[pl.ds(base, eb)], dst_v)
            if not split:
                for i in range(eb // LANES):
                    sl = pl.ds(i * LANES, LANES)
                    src_v[sl] = src_v[sl] + coff
            pltpu.async_copy(sup_hbm.at[src_v], rows_v, sem).wait()
            pltpu.sync_copy(rows_v, acc.at[dst_v], add=True)
            return carry

        lax.fori_loop(0, nb, block, 0)
        plsc.subcore_barrier()

        obase = pl.multiple_of(cid * N + row0, 8)
        pltpu.sync_copy(acc.at[pl.ds(row0, RPT)], out_hbm.at[pl.ds(obase, RPT)])

        @pl.when(last)
        def _():
            pltpu.sync_copy(
                acc.at[pl.ds(RPT * NTILES, RTAIL)],
                out_hbm.at[pl.ds(pl.multiple_of(cid * N + RPT * NTILES, 8),
                                 RTAIL)])

    return k(sup_flat, src, dst, zeros)


def kernel(x, edge_index, W1, W2, W3, Wd1, Wd2):
    src = edge_index[0]
    dst = edge_index[1]
    W23 = jnp.concatenate([W2, W3], axis=1)
    eps = jax.random.normal(jax.random.key(1), (N, H2), dtype=jnp.float32)

    sup1 = _mm(x, W1)
    agg1 = _spmm(sup1, src, dst)
    sup23 = _mm(agg1.reshape(2, N, 128), W23, relu=True)
    agg23p = _spmm(sup23, src, dst, split=True)
    mu, logvar, z = _mlz(agg23p, eps)

    recon_adj = _zzt(z)

    supd1 = _mm(z, Wd1)
    aggd1 = _spmm(supd1, src, dst)
    supd2 = _mm(aggd1.reshape(2, N, 128), Wd2, relu=True)
    xr = _spmm(supd2, src, dst)
    x_rec = xr.reshape(2, N, 128).transpose(1, 0, 2).reshape(N, 256)

    return (recon_adj, mu, logvar, z, x_rec)


# speedup vs baseline: 5.2282x; 1.7777x over previous
import functools

import jax
import jax.numpy as jnp
from jax import lax
from jax.experimental import pallas as pl
from jax.experimental.pallas import tpu as pltpu
from jax.experimental.pallas import tpu_sc as plsc

N = 10000
E = 160000
H2 = 64
LANES = 16
NCORES = 2
NTILES = 16
RPT = 624
RTAIL = N - RPT * NTILES
E_PER_TILE = E // NTILES
EB = 80
NB = E_PER_TILE // EB


def _mm_body_2d(a_ref, w_ref, o_ref, *, relu):
    a = a_ref[...]
    if relu:
        a = jnp.maximum(a, 0.0)
    o_ref[...] = jnp.dot(a, w_ref[0], preferred_element_type=jnp.float32)


def _mm_body_3d(a_ref, w_ref, o_ref, *, relu, ci, cw_in):
    acc = None
    for i in range(ci):
        a = a_ref[i]
        if relu:
            a = jnp.maximum(a, 0.0)
        p = jnp.dot(a, w_ref[0, i * cw_in:(i + 1) * cw_in, :],
                    preferred_element_type=jnp.float32)
        acc = p if acc is None else acc + p
    o_ref[...] = acc


def _mm(a, w, relu=False, cw_out=128, rows=1000):
    k, fo = w.shape
    co = fo // cw_out
    if a.ndim == 2:
        n = a.shape[0]
        body = functools.partial(_mm_body_2d, relu=relu)
        a_spec = pl.BlockSpec((rows, k), lambda c, r: (r, 0))
    else:
        ci, n, cw_in = a.shape
        body = functools.partial(_mm_body_3d, relu=relu, ci=ci, cw_in=cw_in)
        a_spec = pl.BlockSpec((ci, rows, cw_in), lambda c, r: (0, r, 0))
    nr = n // rows
    w3 = w.reshape(k, co, cw_out).transpose(1, 0, 2)
    return pl.pallas_call(
        body,
        grid=(co, nr),
        in_specs=[
            a_spec,
            pl.BlockSpec((1, k, cw_out), lambda c, r: (c, 0, 0)),
        ],
        out_specs=pl.BlockSpec((rows, cw_out), lambda c, r: (c * nr + r, 0)),
        out_shape=jax.ShapeDtypeStruct((co * n, cw_out), jnp.float32),
    )(a, w3)


def _mlz_body(p0_ref, p1_ref, eps_ref, mu_ref, lv_ref, z_ref):
    s = p0_ref[...] + p1_ref[...]
    mu = s[:, :H2]
    lv = s[:, H2:]
    mu_ref[...] = mu
    lv_ref[...] = lv
    z_ref[...] = eps_ref[...] * jnp.exp(lv) + mu


def _mlz(agg23p, eps, rows=1000):
    nr = N // rows
    ospec = pl.BlockSpec((rows, H2), lambda r: (r, 0))
    oshape = jax.ShapeDtypeStruct((N, H2), jnp.float32)
    return pl.pallas_call(
        _mlz_body,
        grid=(nr,),
        in_specs=[
            pl.BlockSpec((rows, 2 * H2), lambda r: (r, 0)),
            pl.BlockSpec((rows, 2 * H2), lambda r: (r + nr, 0)),
            ospec,
        ],
        out_specs=[ospec, ospec, ospec],
        out_shape=[oshape, oshape, oshape],
    )(agg23p, agg23p, eps)


def _zzt_body(a_ref, b_ref, o_ref):
    o_ref[...] = lax.dot_general(
        a_ref[...], b_ref[...], (((1,), (1,)), ((), ())),
        preferred_element_type=jnp.float32)


def _zzt(z, rows=400):
    nr = N // rows
    return pl.pallas_call(
        _zzt_body,
        grid=(nr,),
        in_specs=[
            pl.BlockSpec((rows, H2), lambda i: (i, 0)),
            pl.BlockSpec((N, H2), lambda i: (0, 0)),
        ],
        out_specs=pl.BlockSpec((rows, N), lambda i: (i, 0)),
        out_shape=jax.ShapeDtypeStruct((N, N), jnp.float32),
    )(z, z)


def _spmm(sup_flat, srcz, dst, split=False):
    mesh = plsc.VectorSubcoreMesh(core_axis_name="c", subcore_axis_name="s")
    zeros = jnp.zeros((N, 128), jnp.float32)
    eb = 40 if split else EB
    nb = (E // NCORES // NTILES // eb) if split else NB
    assert nb % 2 == 1

    @functools.partial(
        pl.kernel,
        mesh=mesh,
        out_type=jax.ShapeDtypeStruct((NCORES * N, 128), jnp.float32),
        scratch_types=[
            pltpu.VMEM((2, eb), jnp.int32),
            pltpu.VMEM((2, eb), jnp.int32),
            pltpu.VMEM((2, eb, 128), jnp.float32),
            pltpu.VMEM_SHARED((N, 128), jnp.float32),
            pltpu.SemaphoreType.DMA,
            pltpu.SemaphoreType.DMA,
            pltpu.SemaphoreType.DMA,
            pltpu.SemaphoreType.DMA,
        ],
    )
    def k(sup_hbm, src_hbm, dst_hbm, zer_hbm, out_hbm,
          src_v, dst_v, rows_v, acc, si0, si1, sg0, sg1):
        cid = lax.axis_index("c")
        sid = lax.axis_index("s")
        row0 = pl.multiple_of(sid * RPT, 8)
        if split:
            sbase0 = cid * (E // NCORES) + sid * (E // NCORES // NTILES)
            dbase0 = sbase0
        else:
            sbase0 = cid * E + sid * E_PER_TILE
            dbase0 = sid * E_PER_TILE
        last = sid == NTILES - 1
        sem_i = (si0, si1)
        sem_g = (sg0, sg1)

        pltpu.sync_copy(zer_hbm.at[pl.ds(row0, RPT)], acc.at[pl.ds(row0, RPT)])

        @pl.when(last)
        def _():
            pltpu.sync_copy(zer_hbm.at[pl.ds(RPT * NTILES, RTAIL)],
                            acc.at[pl.ds(RPT * NTILES, RTAIL)])

        plsc.subcore_barrier()

        def idx_start(b, j):
            sb = pl.multiple_of(sbase0 + b * eb, 8)
            db = pl.multiple_of(dbase0 + b * eb, 8)
            pltpu.make_async_copy(
                src_hbm.at[pl.ds(sb, eb)], src_v.at[j], sem_i[j]).start()
            pltpu.make_async_copy(
                dst_hbm.at[pl.ds(db, eb)], dst_v.at[j], sem_i[j]).start()

        def idx_wait(j):
            pltpu.make_async_copy(
                src_hbm.at[pl.ds(0, eb)], src_v.at[j], sem_i[j]).wait()
            pltpu.make_async_copy(
                dst_hbm.at[pl.ds(0, eb)], dst_v.at[j], sem_i[j]).wait()

        def gather_start(j):
            pltpu.make_async_copy(
                sup_hbm.at[src_v.at[j]], rows_v.at[j], sem_g[j]).start()

        def gather_wait(j):
            pltpu.make_async_copy(
                sup_hbm.at[src_v.at[j]], rows_v.at[j], sem_g[j]).wait()

        def scatter(j):
            pltpu.sync_copy(rows_v.at[j], acc.at[dst_v.at[j]], add=True)

        idx_start(0, 0)
        idx_start(1, 1)
        idx_wait(0)
        gather_start(0)

        def pair(g, carry):
            b = 2 * g
            idx_wait(1)
            gather_start(1)
            gather_wait(0)
            scatter(0)

            @pl.when(b + 2 < nb)
            def _():
                idx_start(b + 2, 0)
                idx_wait(0)
                gather_start(0)

            gather_wait(1)
            scatter(1)

            @pl.when(b + 3 < nb)
            def _():
                idx_start(b + 3, 1)

            return carry

        lax.fori_loop(0, nb // 2, pair, 0)
        gather_wait(0)
        scatter(0)
        plsc.subcore_barrier()

        obase = pl.multiple_of(cid * N + row0, 8)
        pltpu.sync_copy(acc.at[pl.ds(row0, RPT)], out_hbm.at[pl.ds(obase, RPT)])

        @pl.when(last)
        def _():
            pltpu.sync_copy(
                acc.at[pl.ds(RPT * NTILES, RTAIL)],
                out_hbm.at[pl.ds(pl.multiple_of(cid * N + RPT * NTILES, 8),
                                 RTAIL)])

    return k(sup_flat, srcz, dst, zeros)


def kernel(x, edge_index, W1, W2, W3, Wd1, Wd2):
    src = edge_index[0]
    dst = edge_index[1]
    srcz = jnp.concatenate([src, src + N])
    W23 = jnp.concatenate([W2, W3], axis=1)
    eps = jax.random.normal(jax.random.key(1), (N, H2), dtype=jnp.float32)

    sup1 = _mm(x, W1)
    agg1 = _spmm(sup1, srcz, dst)
    sup23 = _mm(agg1.reshape(2, N, 128), W23, relu=True)
    agg23p = _spmm(sup23, src, dst, split=True)
    mu, logvar, z = _mlz(agg23p, eps)

    recon_adj = _zzt(z)

    supd1 = _mm(z, Wd1)
    aggd1 = _spmm(supd1, srcz, dst)
    supd2 = _mm(aggd1.reshape(2, N, 128), Wd2, relu=True)
    xr = _spmm(supd2, srcz, dst)
    x_rec = xr.reshape(2, N, 128).transpose(1, 0, 2).reshape(N, 256)

    return (recon_adj, mu, logvar, z, x_rec)


# speedup vs baseline: 5.2344x; 1.0012x over previous
import functools

import jax
import jax.numpy as jnp
from jax import lax
from jax.experimental import pallas as pl
from jax.experimental.pallas import tpu as pltpu
from jax.experimental.pallas import tpu_sc as plsc

N = 10000
E = 160000
H2 = 64
LANES = 16
NCORES = 2
NTILES = 16
RPT = 624
RTAIL = N - RPT * NTILES
E_PER_TILE = E // NTILES
EB = 80
NB = E_PER_TILE // EB


def _mm_body_2d(a_ref, w_ref, o_ref, *, relu):
    a = a_ref[...]
    if relu:
        a = jnp.maximum(a, 0.0)
    o_ref[...] = jnp.dot(a, w_ref[0], preferred_element_type=jnp.float32)


def _mm_body_3d(a_ref, w_ref, o_ref, *, relu, ci, cw_in):
    acc = None
    for i in range(ci):
        a = a_ref[i]
        if relu:
            a = jnp.maximum(a, 0.0)
        p = jnp.dot(a, w_ref[0, i * cw_in:(i + 1) * cw_in, :],
                    preferred_element_type=jnp.float32)
        acc = p if acc is None else acc + p
    o_ref[...] = acc


def _mm(a, w, relu=False, cw_out=128, rows=1000):
    k, fo = w.shape
    co = fo // cw_out
    if a.ndim == 2:
        n = a.shape[0]
        body = functools.partial(_mm_body_2d, relu=relu)
        a_spec = pl.BlockSpec((rows, k), lambda c, r: (r, 0))
    else:
        ci, n, cw_in = a.shape
        body = functools.partial(_mm_body_3d, relu=relu, ci=ci, cw_in=cw_in)
        a_spec = pl.BlockSpec((ci, rows, cw_in), lambda c, r: (0, r, 0))
    nr = n // rows
    w3 = w.reshape(k, co, cw_out).transpose(1, 0, 2)
    return pl.pallas_call(
        body,
        grid=(co, nr),
        in_specs=[
            a_spec,
            pl.BlockSpec((1, k, cw_out), lambda c, r: (c, 0, 0)),
        ],
        out_specs=pl.BlockSpec((rows, cw_out), lambda c, r: (c * nr + r, 0)),
        out_shape=jax.ShapeDtypeStruct((co * n, cw_out), jnp.float32),
    )(a, w3)


def _mlz_body(p0_ref, p1_ref, eps_ref, mu_ref, lv_ref, z_ref):
    s = p0_ref[...] + p1_ref[...]
    mu = s[:, :H2]
    lv = s[:, H2:]
    mu_ref[...] = mu
    lv_ref[...] = lv
    z_ref[...] = eps_ref[...] * jnp.exp(lv) + mu


def _mlz(agg23p, eps, rows=1000):
    nr = N // rows
    ospec = pl.BlockSpec((rows, H2), lambda r: (r, 0))
    oshape = jax.ShapeDtypeStruct((N, H2), jnp.float32)
    return pl.pallas_call(
        _mlz_body,
        grid=(nr,),
        in_specs=[
            pl.BlockSpec((rows, 2 * H2), lambda r: (r, 0)),
            pl.BlockSpec((rows, 2 * H2), lambda r: (r + nr, 0)),
            ospec,
        ],
        out_specs=[ospec, ospec, ospec],
        out_shape=[oshape, oshape, oshape],
    )(agg23p, agg23p, eps)


def _zzt_body(a_ref, b_ref, o_ref):
    o_ref[...] = lax.dot_general(
        a_ref[...], b_ref[...], (((1,), (1,)), ((), ())),
        preferred_element_type=jnp.float32)


def _zzt(z, rows=400):
    nr = N // rows
    return pl.pallas_call(
        _zzt_body,
        grid=(nr,),
        in_specs=[
            pl.BlockSpec((rows, H2), lambda i: (i, 0)),
            pl.BlockSpec((N, H2), lambda i: (0, 0)),
        ],
        out_specs=pl.BlockSpec((rows, N), lambda i: (i, 0)),
        out_shape=jax.ShapeDtypeStruct((N, N), jnp.float32),
    )(z, z)


def _spmm(sup_flat, srcz, dst, split=False):
    mesh = plsc.VectorSubcoreMesh(core_axis_name="c", subcore_axis_name="s")
    zeros = jnp.zeros((N, 128), jnp.float32)
    eb = 40 if split else EB
    nb = (E // NCORES // NTILES // eb) if split else NB
    assert nb % 2 == 1

    @functools.partial(
        pl.kernel,
        mesh=mesh,
        out_type=jax.ShapeDtypeStruct((NCORES * N, 128), jnp.float32),
        scratch_types=[
            pltpu.VMEM((2, eb), jnp.int32),
            pltpu.VMEM((2, eb), jnp.int32),
            pltpu.VMEM((2, eb, 128), jnp.float32),
            pltpu.VMEM_SHARED((N, 128), jnp.float32),
            pltpu.SemaphoreType.DMA,
            pltpu.SemaphoreType.DMA,
            pltpu.SemaphoreType.DMA,
            pltpu.SemaphoreType.DMA,
        ],
    )
    def k(sup_hbm, src_hbm, dst_hbm, zer_hbm, out_hbm,
          src_v, dst_v, rows_v, acc, si0, si1, sg0, sg1):
        cid = lax.axis_index("c")
        sid = lax.axis_index("s")
        row0 = pl.multiple_of(sid * RPT, 8)
        if split:
            sbase0 = cid * (E // NCORES) + sid * (E // NCORES // NTILES)
            dbase0 = sbase0
        else:
            sbase0 = cid * E + sid * E_PER_TILE
            dbase0 = sid * E_PER_TILE
        last = sid == NTILES - 1
        sem_i = (si0, si1)
        sem_g = (sg0, sg1)

        pltpu.sync_copy(zer_hbm.at[pl.ds(row0, RPT)], acc.at[pl.ds(row0, RPT)])

        @pl.when(last)
        def _():
            pltpu.sync_copy(zer_hbm.at[pl.ds(RPT * NTILES, RTAIL)],
                            acc.at[pl.ds(RPT * NTILES, RTAIL)])

        plsc.subcore_barrier()

        def idx_start(b, j):
            sb = pl.multiple_of(sbase0 + b * eb, 8)
            db = pl.multiple_of(dbase0 + b * eb, 8)
            pltpu.make_async_copy(
                src_hbm.at[pl.ds(sb, eb)], src_v.at[j], sem_i[j]).start()
            pltpu.make_async_copy(
                dst_hbm.at[pl.ds(db, eb)], dst_v.at[j], sem_i[j]).start()

        def idx_wait(j):
            pltpu.make_async_copy(
                src_hbm.at[pl.ds(0, eb)], src_v.at[j], sem_i[j]).wait()
            pltpu.make_async_copy(
                dst_hbm.at[pl.ds(0, eb)], dst_v.at[j], sem_i[j]).wait()

        def gather_start(j):
            pltpu.make_async_copy(
                sup_hbm.at[src_v.at[j]], rows_v.at[j], sem_g[j]).start()

        def gather_wait(j):
            pltpu.make_async_copy(
                sup_hbm.at[src_v.at[j]], rows_v.at[j], sem_g[j]).wait()

        def scatter(j):
            pltpu.sync_copy(rows_v.at[j], acc.at[dst_v.at[j]], add=True)

        idx_start(0, 0)
        idx_start(1, 1)
        idx_wait(0)
        gather_start(0)

        def pair(g, carry):
            b = 2 * g
            idx_wait(1)
            gather_start(1)
            gather_wait(0)
            scatter(0)

            @pl.when(b + 2 < nb)
            def _():
                idx_start(b + 2, 0)
                idx_wait(0)
                gather_start(0)

            gather_wait(1)
            scatter(1)

            @pl.when(b + 3 < nb)
            def _():
                idx_start(b + 3, 1)

            return carry

        lax.fori_loop(0, nb // 2, pair, 0)
        gather_wait(0)
        scatter(0)
        plsc.subcore_barrier()

        obase = pl.multiple_of(cid * N + row0, 8)
        pltpu.sync_copy(acc.at[pl.ds(row0, RPT)], out_hbm.at[pl.ds(obase, RPT)])

        @pl.when(last)
        def _():
            pltpu.sync_copy(
                acc.at[pl.ds(RPT * NTILES, RTAIL)],
                out_hbm.at[pl.ds(pl.multiple_of(cid * N + RPT * NTILES, 8),
                                 RTAIL)])

    return k(sup_flat, srcz, dst, zeros)


def kernel(x, edge_index, W1, W2, W3, Wd1, Wd2):
    src = edge_index[0]
    dst = edge_index[1]
    srcz = jnp.concatenate([src, src + N])
    W23 = jnp.concatenate([W2, W3], axis=1)
    eps = jax.random.normal(jax.random.key(1), (N, H2), dtype=jnp.float32)

    sup1 = _mm(x, W1)
    agg1 = _spmm(sup1, srcz, dst)
    sup23 = _mm(agg1.reshape(2, N, 128), W23, relu=True)
    agg23p = _spmm(sup23, src, dst, split=True)
    mu, logvar, z = _mlz(agg23p, eps)

    supd1 = _mm(z, Wd1)
    aggd1 = _spmm(supd1, srcz, dst)
    recon_adj = _zzt(z)
    supd2 = _mm(aggd1.reshape(2, N, 128), Wd2, relu=True)
    xr = _spmm(supd2, srcz, dst)
    x_rec = xr.reshape(2, N, 128).transpose(1, 0, 2).reshape(N, 256)

    return (recon_adj, mu, logvar, z, x_rec)


# speedup vs baseline: 6.0701x; 1.1597x over previous
import functools

import jax
import jax.numpy as jnp
from jax import lax
from jax.experimental import pallas as pl
from jax.experimental.pallas import tpu as pltpu
from jax.experimental.pallas import tpu_sc as plsc

N = 10000
E = 160000
H2 = 64
LANES = 16
NCORES = 2
NTILES = 16
RPT = 624
RTAIL = N - RPT * NTILES
E_PER_TILE = E // NTILES
EB = 80
NB = E_PER_TILE // EB
EPAD = 2 * NTILES * 63 * EB
EP_TILE = EPAD // NCORES // NTILES
NDUMP = 16


def _mm_body_2d(a_ref, w_ref, o_ref, *, relu):
    a = a_ref[...]
    if relu:
        a = jnp.maximum(a, 0.0)
    o_ref[...] = jnp.dot(a, w_ref[0], preferred_element_type=jnp.float32)


def _mm_body_3d(a_ref, w_ref, o_ref, *, relu, ci, cw_in):
    acc = None
    for i in range(ci):
        a = a_ref[i]
        if relu:
            a = jnp.maximum(a, 0.0)
        p = jnp.dot(a, w_ref[0, i * cw_in:(i + 1) * cw_in, :],
                    preferred_element_type=jnp.float32)
        acc = p if acc is None else acc + p
    o_ref[...] = acc


def _mm(a, w, relu=False, cw_out=128, rows=1000):
    k, fo = w.shape
    co = fo // cw_out
    if a.ndim == 2:
        n = a.shape[0]
        body = functools.partial(_mm_body_2d, relu=relu)
        a_spec = pl.BlockSpec((rows, k), lambda c, r: (r, 0))
    else:
        ci, n, cw_in = a.shape
        body = functools.partial(_mm_body_3d, relu=relu, ci=ci, cw_in=cw_in)
        a_spec = pl.BlockSpec((ci, rows, cw_in), lambda c, r: (0, r, 0))
    nr = n // rows
    w3 = w.reshape(k, co, cw_out).transpose(1, 0, 2)
    return pl.pallas_call(
        body,
        grid=(co, nr),
        in_specs=[
            a_spec,
            pl.BlockSpec((1, k, cw_out), lambda c, r: (c, 0, 0)),
        ],
        out_specs=pl.BlockSpec((rows, cw_out), lambda c, r: (c * nr + r, 0)),
        out_shape=jax.ShapeDtypeStruct((co * n, cw_out), jnp.float32),
    )(a, w3)


def _mlz_body(p0_ref, p1_ref, eps_ref, mu_ref, lv_ref, z_ref, zp_ref):
    s = p0_ref[...] + p1_ref[...]
    mu = s[:, :H2]
    lv = s[:, H2:]
    mu_ref[...] = mu
    lv_ref[...] = lv
    z = eps_ref[...] * jnp.exp(lv) + mu
    z_ref[...] = z
    zp_ref[...] = jnp.concatenate([z, jnp.zeros_like(z)], axis=1)


def _mlz(agg23p, eps, rows=1000):
    nr = N // rows
    ospec = pl.BlockSpec((rows, H2), lambda r: (r, 0))
    oshape = jax.ShapeDtypeStruct((N, H2), jnp.float32)
    pspec = pl.BlockSpec((rows, 2 * H2), lambda r: (r, 0))
    return pl.pallas_call(
        _mlz_body,
        grid=(nr,),
        in_specs=[
            pspec,
            pl.BlockSpec((rows, 2 * H2), lambda r: (r + nr, 0)),
            ospec,
        ],
        out_specs=[ospec, ospec, ospec, pspec],
        out_shape=[oshape, oshape, oshape,
                   jax.ShapeDtypeStruct((N, 2 * H2), jnp.float32)],
    )(agg23p, agg23p, eps)


def _mm2_body(p0_ref, p1_ref, w1_ref, w2_ref, o_ref):
    az = (p0_ref[...] + p1_ref[...])[:, :H2]
    hd = jnp.maximum(jnp.dot(az, w1_ref[...],
                             preferred_element_type=jnp.float32), 0.0)
    o_ref[...] = jnp.dot(hd, w2_ref[0], preferred_element_type=jnp.float32)


def _mm2(azp, w1, w2, rows=1000):
    nr = N // rows
    w23 = w2.reshape(w2.shape[0], 2, 128).transpose(1, 0, 2)
    return pl.pallas_call(
        _mm2_body,
        grid=(2, nr),
        in_specs=[
            pl.BlockSpec((rows, 2 * H2), lambda c, r: (r, 0)),
            pl.BlockSpec((rows, 2 * H2), lambda c, r: (r + nr, 0)),
            pl.BlockSpec((H2, w1.shape[1]), lambda c, r: (0, 0)),
            pl.BlockSpec((1, w2.shape[0], 128), lambda c, r: (c, 0, 0)),
        ],
        out_specs=pl.BlockSpec((rows, 128), lambda c, r: (c * nr + r, 0)),
        out_shape=jax.ShapeDtypeStruct((2 * N, 128), jnp.float32),
    )(azp, azp, w1, w23)


def _zzt_body(a_ref, b_ref, o_ref):
    o_ref[...] = lax.dot_general(
        a_ref[...], b_ref[...], (((1,), (1,)), ((), ())),
        preferred_element_type=jnp.float32)


def _zzt(z, rows=400):
    nr = N // rows
    return pl.pallas_call(
        _zzt_body,
        grid=(nr,),
        in_specs=[
            pl.BlockSpec((rows, H2), lambda i: (i, 0)),
            pl.BlockSpec((N, H2), lambda i: (0, 0)),
        ],
        out_specs=pl.BlockSpec((rows, N), lambda i: (i, 0)),
        out_shape=jax.ShapeDtypeStruct((N, N), jnp.float32),
    )(z, z)


def _spmm(sup_flat, srcz, dst, split=False):
    mesh = plsc.VectorSubcoreMesh(core_axis_name="c", subcore_axis_name="s")
    zeros = jnp.zeros((N + NDUMP, 128), jnp.float32)
    eb = EB
    nb = EP_TILE // eb if split else NB
    assert nb % 2 == 1

    @functools.partial(
        pl.kernel,
        mesh=mesh,
        out_type=jax.ShapeDtypeStruct((NCORES * N, 128), jnp.float32),
        scratch_types=[
            pltpu.VMEM((2, eb), jnp.int32),
            pltpu.VMEM((2, eb), jnp.int32),
            pltpu.VMEM((2, eb, 128), jnp.float32),
            pltpu.VMEM_SHARED((N + NDUMP, 128), jnp.float32),
            pltpu.SemaphoreType.DMA,
            pltpu.SemaphoreType.DMA,
            pltpu.SemaphoreType.DMA,
            pltpu.SemaphoreType.DMA,
        ],
    )
    def k(sup_hbm, src_hbm, dst_hbm, zer_hbm, out_hbm,
          src_v, dst_v, rows_v, acc, si0, si1, sg0, sg1):
        cid = lax.axis_index("c")
        sid = lax.axis_index("s")
        row0 = pl.multiple_of(sid * RPT, 8)
        if split:
            sbase0 = cid * (EPAD // NCORES) + sid * EP_TILE
            dbase0 = sbase0
        else:
            sbase0 = cid * E + sid * E_PER_TILE
            dbase0 = sid * E_PER_TILE
        last = sid == NTILES - 1
        sem_i = (si0, si1)
        sem_g = (sg0, sg1)

        pltpu.sync_copy(zer_hbm.at[pl.ds(row0, RPT)], acc.at[pl.ds(row0, RPT)])

        @pl.when(last)
        def _():
            pltpu.sync_copy(zer_hbm.at[pl.ds(RPT * NTILES, RTAIL + NDUMP)],
                            acc.at[pl.ds(RPT * NTILES, RTAIL + NDUMP)])

        plsc.subcore_barrier()

        def idx_start(b, j):
            sb = pl.multiple_of(sbase0 + b * eb, 8)
            db = pl.multiple_of(dbase0 + b * eb, 8)
            pltpu.make_async_copy(
                src_hbm.at[pl.ds(sb, eb)], src_v.at[j], sem_i[j]).start()
            pltpu.make_async_copy(
                dst_hbm.at[pl.ds(db, eb)], dst_v.at[j], sem_i[j]).start()

        def idx_wait(j):
            pltpu.make_async_copy(
                src_hbm.at[pl.ds(0, eb)], src_v.at[j], sem_i[j]).wait()
            pltpu.make_async_copy(
                dst_hbm.at[pl.ds(0, eb)], dst_v.at[j], sem_i[j]).wait()

        def gather_start(j):
            pltpu.make_async_copy(
                sup_hbm.at[src_v.at[j]], rows_v.at[j], sem_g[j]).start()

        def gather_wait(j):
            pltpu.make_async_copy(
                sup_hbm.at[src_v.at[j]], rows_v.at[j], sem_g[j]).wait()

        def scatter(j):
            pltpu.sync_copy(rows_v.at[j], acc.at[dst_v.at[j]], add=True)

        idx_start(0, 0)
        idx_start(1, 1)
        idx_wait(0)
        gather_start(0)

        def pair(g, carry):
            b = 2 * g
            idx_wait(1)
            gather_start(1)
            gather_wait(0)
            scatter(0)

            @pl.when(b + 2 < nb)
            def _():
                idx_start(b + 2, 0)
                idx_wait(0)
                gather_start(0)

            gather_wait(1)
            scatter(1)

            @pl.when(b + 3 < nb)
            def _():
                idx_start(b + 3, 1)

            return carry

        lax.fori_loop(0, nb // 2, pair, 0)
        gather_wait(0)
        scatter(0)
        plsc.subcore_barrier()

        obase = pl.multiple_of(cid * N + row0, 8)
        pltpu.sync_copy(acc.at[pl.ds(row0, RPT)], out_hbm.at[pl.ds(obase, RPT)])

        @pl.when(last)
        def _():
            pltpu.sync_copy(
                acc.at[pl.ds(RPT * NTILES, RTAIL)],
                out_hbm.at[pl.ds(pl.multiple_of(cid * N + RPT * NTILES, 8),
                                 RTAIL)])

    return k(sup_flat, srcz, dst, zeros)


def kernel(x, edge_index, W1, W2, W3, Wd1, Wd2):
    src = edge_index[0]
    dst = edge_index[1]
    srcz = jnp.concatenate([src, src + N])
    npad = EPAD - E
    pad = jnp.arange(npad, dtype=jnp.int32) % NDUMP
    srcp = jnp.concatenate([src, pad])
    dstp = jnp.concatenate([dst, pad + N])
    W23 = jnp.concatenate([W2, W3], axis=1)
    eps = jax.random.normal(jax.random.key(1), (N, H2), dtype=jnp.float32)

    sup1 = _mm(x, W1)
    agg1 = _spmm(sup1, srcz, dst)
    sup23 = _mm(agg1.reshape(2, N, 128), W23, relu=True)
    agg23p = _spmm(sup23, srcp, dstp, split=True)
    mu, logvar, z, zpad = _mlz(agg23p, eps)

    azp = _spmm(zpad, srcp, dstp, split=True)
    recon_adj = _zzt(z)
    supd2 = _mm2(azp, Wd1, Wd2)
    xr = _spmm(supd2, srcz, dst)
    x_rec = xr.reshape(2, N, 128).transpose(1, 0, 2).reshape(N, 256)

    return (recon_adj, mu, logvar, z, x_rec)


# speedup vs baseline: 6.3269x; 1.0423x over previous
import functools

import jax
import jax.numpy as jnp
from jax import lax
from jax.experimental import pallas as pl
from jax.experimental.pallas import tpu as pltpu
from jax.experimental.pallas import tpu_sc as plsc

N = 10000
E = 160000
H2 = 64
LANES = 16
NCORES = 2
NTILES = 16
RPT = 624
RTAIL = N - RPT * NTILES
E_PER_TILE = E // NTILES
EB = 128
NB = 79
PT_FULL = NB * EB
NB_S = 41
PT_SPLIT = NB_S * EB
NDUMP = 16


def _mm_body_2d(a_ref, w_ref, o_ref, *, relu):
    a = a_ref[...]
    if relu:
        a = jnp.maximum(a, 0.0)
    o_ref[...] = jnp.dot(a, w_ref[0], preferred_element_type=jnp.float32)


def _mm_body_3d(a_ref, w_ref, o_ref, *, relu, ci, cw_in):
    acc = None
    for i in range(ci):
        a = a_ref[i]
        if relu:
            a = jnp.maximum(a, 0.0)
        p = jnp.dot(a, w_ref[0, i * cw_in:(i + 1) * cw_in, :],
                    preferred_element_type=jnp.float32)
        acc = p if acc is None else acc + p
    o_ref[...] = acc


def _mm(a, w, relu=False, cw_out=128, rows=1000):
    k, fo = w.shape
    co = fo // cw_out
    if a.ndim == 2:
        n = a.shape[0]
        body = functools.partial(_mm_body_2d, relu=relu)
        a_spec = pl.BlockSpec((rows, k), lambda c, r: (r, 0))
    else:
        ci, n, cw_in = a.shape
        body = functools.partial(_mm_body_3d, relu=relu, ci=ci, cw_in=cw_in)
        a_spec = pl.BlockSpec((ci, rows, cw_in), lambda c, r: (0, r, 0))
    nr = n // rows
    w3 = w.reshape(k, co, cw_out).transpose(1, 0, 2)
    return pl.pallas_call(
        body,
        grid=(co, nr),
        in_specs=[
            a_spec,
            pl.BlockSpec((1, k, cw_out), lambda c, r: (c, 0, 0)),
        ],
        out_specs=pl.BlockSpec((rows, cw_out), lambda c, r: (c * nr + r, 0)),
        out_shape=jax.ShapeDtypeStruct((co * n, cw_out), jnp.float32),
    )(a, w3)


def _mlz_body(p0_ref, p1_ref, eps_ref, mu_ref, lv_ref, z_ref, zp_ref):
    s = p0_ref[...] + p1_ref[...]
    mu = s[:, :H2]
    lv = s[:, H2:]
    mu_ref[...] = mu
    lv_ref[...] = lv
    z = eps_ref[...] * jnp.exp(lv) + mu
    z_ref[...] = z
    zp_ref[...] = jnp.concatenate([z, jnp.zeros_like(z)], axis=1)


def _mlz(agg23p, eps, rows=1000):
    nr = N // rows
    ospec = pl.BlockSpec((rows, H2), lambda r: (r, 0))
    oshape = jax.ShapeDtypeStruct((N, H2), jnp.float32)
    pspec = pl.BlockSpec((rows, 2 * H2), lambda r: (r, 0))
    return pl.pallas_call(
        _mlz_body,
        grid=(nr,),
        in_specs=[
            pspec,
            pl.BlockSpec((rows, 2 * H2), lambda r: (r + nr, 0)),
            ospec,
        ],
        out_specs=[ospec, ospec, ospec, pspec],
        out_shape=[oshape, oshape, oshape,
                   jax.ShapeDtypeStruct((N, 2 * H2), jnp.float32)],
    )(agg23p, agg23p, eps)


def _mm2_body(p0_ref, p1_ref, w1_ref, w2_ref, o_ref):
    az = (p0_ref[...] + p1_ref[...])[:, :H2]
    hd = jnp.maximum(jnp.dot(az, w1_ref[...],
                             preferred_element_type=jnp.float32), 0.0)
    o_ref[...] = jnp.dot(hd, w2_ref[0], preferred_element_type=jnp.float32)


def _mm2(azp, w1, w2, rows=1000):
    nr = N // rows
    w23 = w2.reshape(w2.shape[0], 2, 128).transpose(1, 0, 2)
    return pl.pallas_call(
        _mm2_body,
        grid=(2, nr),
        in_specs=[
            pl.BlockSpec((rows, 2 * H2), lambda c, r: (r, 0)),
            pl.BlockSpec((rows, 2 * H2), lambda c, r: (r + nr, 0)),
            pl.BlockSpec((H2, w1.shape[1]), lambda c, r: (0, 0)),
            pl.BlockSpec((1, w2.shape[0], 128), lambda c, r: (c, 0, 0)),
        ],
        out_specs=pl.BlockSpec((rows, 128), lambda c, r: (c * nr + r, 0)),
        out_shape=jax.ShapeDtypeStruct((2 * N, 128), jnp.float32),
    )(azp, azp, w1, w23)


def _zzt_body(a_ref, b_ref, o_ref):
    o_ref[...] = lax.dot_general(
        a_ref[...], b_ref[...], (((1,), (1,)), ((), ())),
        preferred_element_type=jnp.float32)


def _zzt(z, rows=400):
    nr = N // rows
    return pl.pallas_call(
        _zzt_body,
        grid=(nr,),
        in_specs=[
            pl.BlockSpec((rows, H2), lambda i: (i, 0)),
            pl.BlockSpec((N, H2), lambda i: (0, 0)),
        ],
        out_specs=pl.BlockSpec((rows, N), lambda i: (i, 0)),
        out_shape=jax.ShapeDtypeStruct((N, N), jnp.float32),
    )(z, z)


def _spmm(sup_flat, srcz, dst, split=False):
    mesh = plsc.VectorSubcoreMesh(core_axis_name="c", subcore_axis_name="s")
    zeros = jnp.zeros((N + NDUMP, 128), jnp.float32)
    eb = EB
    nb = NB_S if split else NB
    assert nb % 2 == 1

    @functools.partial(
        pl.kernel,
        mesh=mesh,
        out_type=jax.ShapeDtypeStruct((NCORES * N, 128), jnp.float32),
        scratch_types=[
            pltpu.VMEM((2, eb), jnp.int32),
            pltpu.VMEM((2, eb), jnp.int32),
            pltpu.VMEM((2, eb, 128), jnp.float32),
            pltpu.VMEM_SHARED((N + NDUMP, 128), jnp.float32),
            pltpu.SemaphoreType.DMA,
            pltpu.SemaphoreType.DMA,
            pltpu.SemaphoreType.DMA,
            pltpu.SemaphoreType.DMA,
        ],
    )
    def k(sup_hbm, src_hbm, dst_hbm, zer_hbm, out_hbm,
          src_v, dst_v, rows_v, acc, si0, si1, sg0, sg1):
        cid = lax.axis_index("c")
        sid = lax.axis_index("s")
        row0 = pl.multiple_of(sid * RPT, 8)
        if split:
            sbase0 = cid * (NTILES * PT_SPLIT) + sid * PT_SPLIT
            dbase0 = sbase0
        else:
            sbase0 = cid * (NTILES * PT_FULL) + sid * PT_FULL
            dbase0 = sid * PT_FULL
        last = sid == NTILES - 1
        sem_i = (si0, si1)
        sem_g = (sg0, sg1)

        pltpu.sync_copy(zer_hbm.at[pl.ds(row0, RPT)], acc.at[pl.ds(row0, RPT)])

        @pl.when(last)
        def _():
            pltpu.sync_copy(zer_hbm.at[pl.ds(RPT * NTILES, RTAIL + NDUMP)],
                            acc.at[pl.ds(RPT * NTILES, RTAIL + NDUMP)])

        plsc.subcore_barrier()

        def idx_start(b, j):
            sb = pl.multiple_of(sbase0 + b * eb, 8)
            db = pl.multiple_of(dbase0 + b * eb, 8)
            pltpu.make_async_copy(
                src_hbm.at[pl.ds(sb, eb)], src_v.at[j], sem_i[j]).start()
            pltpu.make_async_copy(
                dst_hbm.at[pl.ds(db, eb)], dst_v.at[j], sem_i[j]).start()

        def idx_wait(j):
            pltpu.make_async_copy(
                src_hbm.at[pl.ds(0, eb)], src_v.at[j], sem_i[j]).wait()
            pltpu.make_async_copy(
                dst_hbm.at[pl.ds(0, eb)], dst_v.at[j], sem_i[j]).wait()

        def gather_start(j):
            pltpu.make_async_copy(
                sup_hbm.at[src_v.at[j]], rows_v.at[j], sem_g[j]).start()

        def gather_wait(j):
            pltpu.make_async_copy(
                sup_hbm.at[src_v.at[j]], rows_v.at[j], sem_g[j]).wait()

        def scatter(j):
            pltpu.sync_copy(rows_v.at[j], acc.at[dst_v.at[j]], add=True)

        idx_start(0, 0)
        idx_start(1, 1)
        idx_wait(0)
        gather_start(0)

        def pair(g, carry):
            b = 2 * g
            idx_wait(1)
            gather_start(1)
            gather_wait(0)
            scatter(0)

            @pl.when(b + 2 < nb)
            def _():
                idx_start(b + 2, 0)
                idx_wait(0)
                gather_start(0)

            gather_wait(1)
            scatter(1)

            @pl.when(b + 3 < nb)
            def _():
                idx_start(b + 3, 1)

            return carry

        lax.fori_loop(0, nb // 2, pair, 0)
        gather_wait(0)
        scatter(0)
        plsc.subcore_barrier()

        obase = pl.multiple_of(cid * N + row0, 8)
        pltpu.sync_copy(acc.at[pl.ds(row0, RPT)], out_hbm.at[pl.ds(obase, RPT)])

        @pl.when(last)
        def _():
            pltpu.sync_copy(
                acc.at[pl.ds(RPT * NTILES, RTAIL)],
                out_hbm.at[pl.ds(pl.multiple_of(cid * N + RPT * NTILES, 8),
                                 RTAIL)])

    return k(sup_flat, srcz, dst, zeros)


def kernel(x, edge_index, W1, W2, W3, Wd1, Wd2):
    src = edge_index[0]
    dst = edge_index[1]

    def pad_seg(a, old, new, off):
        a2 = a.reshape(-1, old)
        fill = jnp.arange(new - old, dtype=jnp.int32) % NDUMP + off
        padv = jnp.broadcast_to(fill, (a2.shape[0], new - old))
        return jnp.concatenate([a2, padv], axis=1).reshape(-1)

    sseg = pad_seg(src, E_PER_TILE, PT_FULL, 0)
    srcz = jnp.concatenate([sseg, sseg + N])
    dstz = pad_seg(dst, E_PER_TILE, PT_FULL, N)
    srcp = pad_seg(src, E // 32, PT_SPLIT, 0)
    dstp = pad_seg(dst, E // 32, PT_SPLIT, N)
    W23 = jnp.concatenate([W2, W3], axis=1)
    eps = jax.random.normal(jax.random.key(1), (N, H2), dtype=jnp.float32)

    sup1 = _mm(x, W1)
    agg1 = _spmm(sup1, srcz, dstz)
    sup23 = _mm(agg1.reshape(2, N, 128), W23, relu=True)
    agg23p = _spmm(sup23, srcp, dstp, split=True)
    mu, logvar, z, zpad = _mlz(agg23p, eps)

    azp = _spmm(zpad, srcp, dstp, split=True)
    recon_adj = _zzt(z)
    supd2 = _mm2(azp, Wd1, Wd2)
    xr = _spmm(supd2, srcz, dstz)
    x_rec = xr.reshape(2, N, 128).transpose(1, 0, 2).reshape(N, 256)

    return (recon_adj, mu, logvar, z, x_rec)
